# Initial kernel scaffold; baseline (speedup 1.0000x reference)
import jax
import jax.numpy as jnp
from jax.experimental import pallas as pl


def kernel(user_idx, pos_item, neg_item, edge_index, emb_weight):
    raise NotImplementedError("write your pallas kernel here")


# speedup vs baseline: 13.8915x; 13.8915x over previous
import functools

import jax
import jax.numpy as jnp
from jax import lax
from jax.experimental import pallas as pl
from jax.experimental.pallas import tpu as pltpu
from jax.experimental.pallas import tpu_sc as plsc

N_NODES = 50000
N_USERS = 10000
N_EDGES = 800000
D = 64
BATCH = 8192
N_LAYERS = 3
CF_WEIGHT = 1.0
WEIGHT_DECAY = 1e-4

NPAD = 50048
HALF = 25000
N_TRASH = 128
S_ROWS = 25216
CHUNK = 80
NE_PAD = 819200
N_CHUNK_ROWS = NE_PAD // CHUNK

_mesh = plsc.VectorSubcoreMesh(core_axis_name="c", subcore_axis_name="s")


def _fill_zeros(ref, nrows, ncols):
    zero16 = jnp.zeros((16,), jnp.float32)

    def body(i, _):
        for k in range(ncols // 16):
            ref[i, pl.ds(k * 16, 16)] = zero16
        return 0

    lax.fori_loop(0, nrows, body, 0)


ROWS_PER_TILE_A = N_CHUNK_ROWS // 32
GROUPS_A = 10
ROWS_PER_GROUP_A = ROWS_PER_TILE_A // GROUPS_A


@functools.partial(
    pl.kernel,
    out_type=(
        jax.ShapeDtypeStruct((2, NPAD, 16), jnp.float32),
        jax.ShapeDtypeStruct((2, N_CHUNK_ROWS, CHUNK), jnp.int32),
    ),
    mesh=_mesh,
    compiler_params=pltpu.CompilerParams(use_tc_tiling_on_sc=False),
    scratch_types=(
        pltpu.VMEM((ROWS_PER_GROUP_A, CHUNK), jnp.int32),
        pltpu.VMEM((ROWS_PER_GROUP_A, CHUNK), jnp.int32),
        pltpu.VMEM((ROWS_PER_GROUP_A, CHUNK), jnp.int32),
        pltpu.VMEM((CHUNK, 16), jnp.float32),
        pltpu.VMEM((3128, 16), jnp.float32),
        pltpu.VMEM_SHARED((NPAD, 16), jnp.float32),
        pltpu.SemaphoreType.DMA,
    ),
)
def _deg_colmap_kernel(col3, degp, colmap, cbuf, mlow, mhigh, ones, zb,
                       degtab, dsem):
    c = lax.axis_index("c")
    s = lax.axis_index("s")
    wid = s * 2 + c

    _fill_zeros(zb, 3128, 16)
    pltpu.sync_copy(zb, degtab.at[pl.ds(s * 3128, 3128)])
    plsc.subcore_barrier()

    one16 = jnp.full((16,), 1.0, jnp.float32)

    def fill_ones(i, _):
        ones[i] = one16
        return 0

    lax.fori_loop(0, CHUNK, fill_ones, 0)

    iota = lax.broadcasted_iota(jnp.int32, (16,), 0)
    half = jnp.int32(HALF)

    def do_group(r0, nrows):
        pltpu.sync_copy(col3.at[pl.ds(r0, nrows)],
                        cbuf.at[pl.ds(0, nrows)])
        for r in range(nrows):
            for k in range(5):
                v = cbuf[r, pl.ds(k * 16, 16)]
                trash = half + ((iota + ((r * CHUNK + k * 16) % N_TRASH))
                                & jnp.int32(N_TRASH - 1))
                mlow[r, pl.ds(k * 16, 16)] = jnp.where(v < half, v, trash)
                mhigh[r, pl.ds(k * 16, 16)] = jnp.where(v >= half, v - half,
                                                        trash)
            pltpu.async_copy(ones, degtab.at[cbuf.at[r]], dsem, add=True)
        pltpu.sync_copy(mlow.at[pl.ds(0, nrows)],
                        colmap.at[0, pl.ds(r0, nrows), :])
        pltpu.sync_copy(mhigh.at[pl.ds(0, nrows)],
                        colmap.at[1, pl.ds(r0, nrows), :])
        for r in range(nrows):
            pltpu.make_async_copy(ones, degtab.at[cbuf.at[r]], dsem).wait()

    def group_body(g, _):
        do_group(wid * ROWS_PER_TILE_A + g * ROWS_PER_GROUP_A,
                 ROWS_PER_GROUP_A)
        return 0

    lax.fori_loop(0, GROUPS_A, group_body, 0)

    plsc.subcore_barrier()
    pltpu.sync_copy(degtab.at[pl.ds(s * 3128, 3128)],
                    degp.at[c, pl.ds(s * 3128, 3128), :])


def _scale_init_body(degp_ref, emb_ref, dinv_ref, dinv2_ref, y0_ref):
    deg = degp_ref[0, :, 0:1] + degp_ref[1, :, 0:1]
    dinv = jnp.where(deg > 0, lax.rsqrt(jnp.maximum(deg, 1e-12)), 0.0)
    dinv_ref[...] = dinv
    dinv2_ref[...] = dinv * dinv
    y0_ref[...] = emb_ref[...] * dinv


def _scale_init(degp, emb_pad):
    nb = 16
    rb = NPAD // nb
    return pl.pallas_call(
        _scale_init_body,
        grid=(nb,),
        in_specs=[
            pl.BlockSpec((2, rb, 16), lambda r: (0, r, 0)),
            pl.BlockSpec((rb, D), lambda r: (r, 0)),
        ],
        out_specs=[
            pl.BlockSpec((rb, 1), lambda r: (r, 0)),
            pl.BlockSpec((rb, 1), lambda r: (r, 0)),
            pl.BlockSpec((rb, D), lambda r: (r, 0)),
        ],
        out_shape=[
            jax.ShapeDtypeStruct((NPAD, 1), jnp.float32),
            jax.ShapeDtypeStruct((NPAD, 1), jnp.float32),
            jax.ShapeDtypeStruct((NPAD, D), jnp.float32),
        ],
    )(degp, emb_pad)


ROWS_PER_TILE_C = N_CHUNK_ROWS // 16
NBUF = 2
SUP = 8
GPS = SUP // NBUF
NSUP = ROWS_PER_TILE_C // SUP
NG = ROWS_PER_TILE_C // NBUF
STRIPE = S_ROWS // 16


@functools.partial(
    pl.kernel,
    out_type=jax.ShapeDtypeStruct((2, S_ROWS, D), jnp.float32),
    mesh=_mesh,
    compiler_params=pltpu.CompilerParams(use_tc_tiling_on_sc=False),
    scratch_types=(
        pltpu.VMEM((2, NBUF * CHUNK, D), jnp.float32),
        pltpu.VMEM((2, SUP, CHUNK), jnp.int32),
        pltpu.VMEM((2, SUP, CHUNK), jnp.int32),
        pltpu.VMEM_SHARED((S_ROWS, D), jnp.float32),
        pltpu.SemaphoreType.DMA,
        pltpu.SemaphoreType.DMA,
        pltpu.SemaphoreType.DMA,
        pltpu.SemaphoreType.DMA,
    ),
)
def _layer_kernel(y, rowidx3, colmap, sout, rows, idxr, idxc, stab,
                  gsem, ssem, isem0, isem1):
    c = lax.axis_index("c")
    s = lax.axis_index("s")
    base_row = s * ROWS_PER_TILE_C

    zrows = NBUF * CHUNK
    _fill_zeros(rows.at[0], zrows, D)

    def zbody(k, _):
        pltpu.sync_copy(rows.at[0], stab.at[pl.ds(s * STRIPE + k * zrows,
                                                  zrows)])
        return 0

    lax.fori_loop(0, STRIPE // zrows, zbody, 0)
    rem = STRIPE % zrows
    pltpu.sync_copy(rows.at[0, pl.ds(0, rem), :],
                    stab.at[pl.ds(s * STRIPE + STRIPE - rem, rem)])
    plsc.subcore_barrier()

    isems = (isem0, isem1)

    def fire_super(u, parity):
        pltpu.async_copy(rowidx3.at[pl.ds(base_row + u * SUP, SUP)],
                         idxr.at[parity], isems[parity])
        pltpu.async_copy(colmap.at[c, pl.ds(base_row + u * SUP, SUP), :],
                         idxc.at[parity], isems[parity])

    def drain_super(parity):
        pltpu.make_async_copy(rowidx3.at[pl.ds(0, SUP)],
                              idxr.at[parity], isems[parity]).wait()
        pltpu.make_async_copy(colmap.at[0, pl.ds(0, SUP), :],
                              idxc.at[parity], isems[parity]).wait()

    def fire_gathers(q, p, r0):
        for b in range(NBUF):
            pltpu.async_copy(y.at[idxr.at[p, r0 + b]],
                             rows.at[q, pl.ds(b * CHUNK, CHUNK), :],
                             gsem)

    def drain_gathers(q, p, r0):
        for b in range(NBUF):
            pltpu.make_async_copy(
                y.at[idxr.at[p, r0 + b]],
                rows.at[q, pl.ds(b * CHUNK, CHUNK), :], gsem).wait()

    def fire_scatters(q, p, r0):
        for b in range(NBUF):
            pltpu.async_copy(rows.at[q, pl.ds(b * CHUNK, CHUNK), :],
                             stab.at[idxc.at[p, r0 + b]], ssem, add=True)

    def drain_scatters(q, p, r0):
        for b in range(NBUF):
            pltpu.make_async_copy(
                rows.at[q, pl.ds(b * CHUNK, CHUNK), :],
                stab.at[idxc.at[p, r0 + b]], ssem).wait()

    def grp(j):
        return (j % 2, (j // 4) % 2, (j % 4) * NBUF)

    fire_super(0, 0)
    fire_super(1, 1)
    drain_super(0)
    fire_gathers(0, 0, 0)

    def body(up, _):
        for j in range(8):
            q, p, r0 = grp(j)
            drain_gathers(q, p, r0)
            if j == 0:
                @pl.when(up > 0)
                def _():
                    drain_scatters(1 - q, 1, 3 * NBUF)
                    fire_super(up * 2 + 1, 1)
            elif j == 4:
                qp, pp, rp = grp(3)
                drain_scatters(qp, pp, rp)

                @pl.when(up < NSUP // 2 - 1)
                def _():
                    fire_super(up * 2 + 2, 0)
            else:
                qp, pp, rp = grp(j - 1)
                drain_scatters(qp, pp, rp)
            qn, pn, rn = grp((j + 1) % 8)
            if j == 7:
                @pl.when(up < NSUP // 2 - 1)
                def _():
                    drain_super(0)
                    fire_gathers(qn, pn, rn)
            else:
                if j == 3:
                    drain_super(1)
                fire_gathers(qn, pn, rn)
            fire_scatters(q, p, r0)
        return 0

    lax.fori_loop(0, NSUP // 2, body, 0)

    q, p, r0 = grp(7)
    drain_scatters(q, p, r0)

    plsc.subcore_barrier()
    pltpu.sync_copy(stab.at[pl.ds(s * STRIPE, STRIPE)],
                    sout.at[c, pl.ds(s * STRIPE, STRIPE), :])


def _scale_layer_body(s_ref, dinv_ref, dinv2_ref, accp_ref, y_ref, acc_ref):
    sv = s_ref[0]
    y_ref[...] = sv * dinv2_ref[...]
    acc_ref[...] = accp_ref[...] + sv * dinv_ref[...]


def _scale_layer(sout, dinv, dinv2, acc_prev):
    rb = 1000
    nb = HALF // rb
    return pl.pallas_call(
        _scale_layer_body,
        grid=(2, nb),
        in_specs=[
            pl.BlockSpec((1, rb, D), lambda c, r: (c, r, 0)),
            pl.BlockSpec((rb, 1), lambda c, r: (c * 25 + r, 0)),
            pl.BlockSpec((rb, 1), lambda c, r: (c * 25 + r, 0)),
            pl.BlockSpec((rb, D), lambda c, r: (c * 25 + r, 0)),
        ],
        out_specs=[
            pl.BlockSpec((rb, D), lambda c, r: (c * 25 + r, 0)),
            pl.BlockSpec((rb, D), lambda c, r: (c * 25 + r, 0)),
        ],
        out_shape=[
            jax.ShapeDtypeStruct((NPAD, D), jnp.float32),
            jax.ShapeDtypeStruct((NPAD, D), jnp.float32),
        ],
    )(sout, dinv, dinv2, acc_prev)


PER_TILE_B = BATCH // 32


@functools.partial(
    pl.kernel,
    out_type=(
        jax.ShapeDtypeStruct((3, BATCH, D), jnp.float32),
        jax.ShapeDtypeStruct((3, BATCH, D), jnp.float32),
    ),
    mesh=_mesh,
    compiler_params=pltpu.CompilerParams(use_tc_tiling_on_sc=False),
    scratch_types=(
        pltpu.VMEM((PER_TILE_B,), jnp.int32),
        pltpu.VMEM((PER_TILE_B, D), jnp.float32),
        pltpu.VMEM((PER_TILE_B, D), jnp.float32),
        pltpu.SemaphoreType.DMA,
    ),
)
def _batch_gather_kernel(acc, emb, idxs, accg, egog, idxb, rbuf, ebuf, sem):
    c = lax.axis_index("c")
    s = lax.axis_index("s")
    wid = s * 2 + c
    for a in range(3):
        pltpu.sync_copy(idxs.at[a, pl.ds(wid * PER_TILE_B, PER_TILE_B)], idxb)
        for j in range(2):
            pltpu.async_copy(acc.at[idxb.at[pl.ds(j * 128, 128)]],
                             rbuf.at[pl.ds(j * 128, 128), :], sem)
            pltpu.async_copy(emb.at[idxb.at[pl.ds(j * 128, 128)]],
                             ebuf.at[pl.ds(j * 128, 128), :], sem)
        for j in range(2):
            pltpu.make_async_copy(acc.at[idxb.at[pl.ds(j * 128, 128)]],
                                  rbuf.at[pl.ds(j * 128, 128), :], sem).wait()
            pltpu.make_async_copy(emb.at[idxb.at[pl.ds(j * 128, 128)]],
                                  ebuf.at[pl.ds(j * 128, 128), :], sem).wait()
        pltpu.sync_copy(rbuf, accg.at[a, pl.ds(wid * PER_TILE_B, PER_TILE_B), :])
        pltpu.sync_copy(ebuf, egog.at[a, pl.ds(wid * PER_TILE_B, PER_TILE_B), :])


def _loss_body(accg_ref, egog_ref, out_ref):
    u = accg_ref[0]
    p = accg_ref[1]
    n = accg_ref[2]
    pos = jnp.sum(u * p, axis=-1)
    neg = jnp.sum(u * n, axis=-1)
    x = (neg - pos) * 0.0625
    cf = jnp.mean(jnp.maximum(x, 0.0) + jnp.log1p(jnp.exp(-jnp.abs(x))))
    e = egog_ref[...]
    reg = 0.5 * jnp.sum(e * e) / float(BATCH)
    out_ref[...] = jnp.reshape(CF_WEIGHT * cf + reg * WEIGHT_DECAY, (1, 1))


def _loss(accg, egog):
    return pl.pallas_call(
        _loss_body,
        out_shape=jax.ShapeDtypeStruct((1, 1), jnp.float32),
    )(accg, egog)


def kernel(user_idx, pos_item, neg_item, edge_index, emb_weight):
    edge_index = edge_index.astype(jnp.int32)
    npad_e = NE_PAD - N_EDGES
    fr = (jnp.arange(npad_e, dtype=jnp.int32) * 37) % N_NODES
    fc = N_NODES + (jnp.arange(npad_e, dtype=jnp.int32) % (NPAD - N_NODES))
    row3 = jnp.concatenate([edge_index[0], fr]).reshape(N_CHUNK_ROWS, CHUNK)
    col3 = jnp.concatenate([edge_index[1], fc]).reshape(N_CHUNK_ROWS, CHUNK)
    emb_pad = jnp.pad(emb_weight, ((0, NPAD - N_NODES), (0, 0)))
    idxs = jnp.stack([user_idx, pos_item, neg_item]).astype(jnp.int32)

    degp, colmap = _deg_colmap_kernel(col3)
    dinv, dinv2, y = _scale_init(degp, emb_pad)
    acc = emb_pad
    for _ in range(N_LAYERS):
        sout = _layer_kernel(y, row3, colmap)
        y, acc = _scale_layer(sout, dinv, dinv2, acc)

    accg, egog = _batch_gather_kernel(acc, emb_weight, idxs)
    loss = _loss(accg, egog)
    return jnp.reshape(loss, ())


# speedup vs baseline: 14.2372x; 1.0249x over previous
import functools

import jax
import jax.numpy as jnp
from jax import lax
from jax.experimental import pallas as pl
from jax.experimental.pallas import tpu as pltpu
from jax.experimental.pallas import tpu_sc as plsc

N_NODES = 50000
N_USERS = 10000
N_EDGES = 800000
D = 64
BATCH = 8192
N_LAYERS = 3
CF_WEIGHT = 1.0
WEIGHT_DECAY = 1e-4

NPAD = 50048
DH = 32
CHUNK = 80
NE_PAD = 819200
N_CHUNK_ROWS = NE_PAD // CHUNK

_mesh = plsc.VectorSubcoreMesh(core_axis_name="c", subcore_axis_name="s")


def _fill_zeros(ref, nrows, ncols):
    zero16 = jnp.zeros((16,), jnp.float32)

    def body(i, _):
        for k in range(ncols // 16):
            ref[i, pl.ds(k * 16, 16)] = zero16
        return 0

    lax.fori_loop(0, nrows, body, 0)


ROWS_PER_TILE_A = N_CHUNK_ROWS // 32
GROUPS_A = 10
ROWS_PER_GROUP_A = ROWS_PER_TILE_A // GROUPS_A


@functools.partial(
    pl.kernel,
    out_type=jax.ShapeDtypeStruct((2, NPAD, 16), jnp.float32),
    mesh=_mesh,
    compiler_params=pltpu.CompilerParams(use_tc_tiling_on_sc=False),
    scratch_types=(
        pltpu.VMEM((ROWS_PER_GROUP_A, CHUNK), jnp.int32),
        pltpu.VMEM((CHUNK, 16), jnp.float32),
        pltpu.VMEM((3128, 16), jnp.float32),
        pltpu.VMEM_SHARED((NPAD, 16), jnp.float32),
        pltpu.SemaphoreType.DMA,
    ),
)
def _deg_kernel(col3, degp, cbuf, ones, zb, degtab, dsem):
    c = lax.axis_index("c")
    s = lax.axis_index("s")
    wid = s * 2 + c

    _fill_zeros(zb, 3128, 16)
    pltpu.sync_copy(zb, degtab.at[pl.ds(s * 3128, 3128)])
    plsc.subcore_barrier()

    one16 = jnp.full((16,), 1.0, jnp.float32)

    def fill_ones(i, _):
        ones[i] = one16
        return 0

    lax.fori_loop(0, CHUNK, fill_ones, 0)

    def do_group(r0, nrows):
        pltpu.sync_copy(col3.at[pl.ds(r0, nrows)],
                        cbuf.at[pl.ds(0, nrows)])
        for r in range(nrows):
            pltpu.async_copy(ones, degtab.at[cbuf.at[r]], dsem, add=True)
        for r in range(nrows):
            pltpu.make_async_copy(ones, degtab.at[cbuf.at[r]], dsem).wait()

    def group_body(g, _):
        do_group(wid * ROWS_PER_TILE_A + g * ROWS_PER_GROUP_A,
                 ROWS_PER_GROUP_A)
        return 0

    lax.fori_loop(0, GROUPS_A, group_body, 0)

    plsc.subcore_barrier()
    pltpu.sync_copy(degtab.at[pl.ds(s * 3128, 3128)],
                    degp.at[c, pl.ds(s * 3128, 3128), :])


def _scale_init_body(degp_ref, emb_ref, dinv_ref, dinv2_ref, y0_ref,
                     acc0_ref):
    deg = degp_ref[0, :, 0:1] + degp_ref[1, :, 0:1]
    dinv = jnp.where(deg > 0, lax.rsqrt(jnp.maximum(deg, 1e-12)), 0.0)
    dinv_ref[...] = dinv
    dinv2_ref[...] = dinv * dinv
    ev = emb_ref[...]
    for h in range(2):
        y0_ref[h] = ev[:, h * DH:(h + 1) * DH] * dinv
        acc0_ref[h] = ev[:, h * DH:(h + 1) * DH]


def _scale_init(degp, emb_pad):
    nb = 16
    rb = NPAD // nb
    return pl.pallas_call(
        _scale_init_body,
        grid=(nb,),
        in_specs=[
            pl.BlockSpec((2, rb, 16), lambda r: (0, r, 0)),
            pl.BlockSpec((rb, D), lambda r: (r, 0)),
        ],
        out_specs=[
            pl.BlockSpec((rb, 1), lambda r: (r, 0)),
            pl.BlockSpec((rb, 1), lambda r: (r, 0)),
            pl.BlockSpec((2, rb, DH), lambda r: (0, r, 0)),
            pl.BlockSpec((2, rb, DH), lambda r: (0, r, 0)),
        ],
        out_shape=[
            jax.ShapeDtypeStruct((NPAD, 1), jnp.float32),
            jax.ShapeDtypeStruct((NPAD, 1), jnp.float32),
            jax.ShapeDtypeStruct((2, NPAD, DH), jnp.float32),
            jax.ShapeDtypeStruct((2, NPAD, DH), jnp.float32),
        ],
    )(degp, emb_pad)


ROWS_PER_TILE_C = N_CHUNK_ROWS // 16
NBUF = 2
SUP = 8
GPS = SUP // NBUF
NSUP = ROWS_PER_TILE_C // SUP
NG = ROWS_PER_TILE_C // NBUF
STRIPE = NPAD // 16


@functools.partial(
    pl.kernel,
    out_type=jax.ShapeDtypeStruct((2, NPAD, DH), jnp.float32),
    mesh=_mesh,
    compiler_params=pltpu.CompilerParams(use_tc_tiling_on_sc=False),
    scratch_types=(
        pltpu.VMEM((2, NBUF * CHUNK, DH), jnp.float32),
        pltpu.VMEM((2, SUP, CHUNK), jnp.int32),
        pltpu.VMEM((2, SUP, CHUNK), jnp.int32),
        pltpu.VMEM_SHARED((NPAD, DH), jnp.float32),
        pltpu.SemaphoreType.DMA,
        pltpu.SemaphoreType.DMA,
        pltpu.SemaphoreType.DMA,
        pltpu.SemaphoreType.DMA,
    ),
)
def _layer_kernel(ycat, ridx2, col3, sout, rows, idxr, idxc, stab,
                  gsem, ssem, isem0, isem1):
    c = lax.axis_index("c")
    s = lax.axis_index("s")
    base_row = s * ROWS_PER_TILE_C

    zrows = NBUF * CHUNK
    _fill_zeros(rows.at[0], zrows, DH)

    def zbody(k, _):
        pltpu.sync_copy(rows.at[0], stab.at[pl.ds(s * STRIPE + k * zrows,
                                                  zrows)])
        return 0

    lax.fori_loop(0, STRIPE // zrows, zbody, 0)
    rem = STRIPE % zrows
    pltpu.sync_copy(rows.at[0, pl.ds(0, rem), :],
                    stab.at[pl.ds(s * STRIPE + STRIPE - rem, rem)])
    plsc.subcore_barrier()

    isems = (isem0, isem1)

    def fire_super(u, parity):
        pltpu.async_copy(ridx2.at[c, pl.ds(base_row + u * SUP, SUP), :],
                         idxr.at[parity], isems[parity])
        pltpu.async_copy(col3.at[pl.ds(base_row + u * SUP, SUP)],
                         idxc.at[parity], isems[parity])

    def drain_super(parity):
        pltpu.make_async_copy(ridx2.at[0, pl.ds(0, SUP), :],
                              idxr.at[parity], isems[parity]).wait()
        pltpu.make_async_copy(col3.at[pl.ds(0, SUP)],
                              idxc.at[parity], isems[parity]).wait()

    def fire_gathers(q, p, r0):
        for b in range(NBUF):
            pltpu.async_copy(ycat.at[idxr.at[p, r0 + b]],
                             rows.at[q, pl.ds(b * CHUNK, CHUNK), :],
                             gsem)

    def drain_gathers(q, p, r0):
        for b in range(NBUF):
            pltpu.make_async_copy(
                ycat.at[idxr.at[p, r0 + b]],
                rows.at[q, pl.ds(b * CHUNK, CHUNK), :], gsem).wait()

    def fire_scatters(q, p, r0):
        for b in range(NBUF):
            pltpu.async_copy(rows.at[q, pl.ds(b * CHUNK, CHUNK), :],
                             stab.at[idxc.at[p, r0 + b]], ssem, add=True)

    def drain_scatters(q, p, r0):
        for b in range(NBUF):
            pltpu.make_async_copy(
                rows.at[q, pl.ds(b * CHUNK, CHUNK), :],
                stab.at[idxc.at[p, r0 + b]], ssem).wait()

    def grp(j):
        return (j % 2, (j // 4) % 2, (j % 4) * NBUF)

    fire_super(0, 0)
    fire_super(1, 1)
    drain_super(0)
    fire_gathers(0, 0, 0)

    def body(up, _):
        for j in range(8):
            q, p, r0 = grp(j)
            drain_gathers(q, p, r0)
            if j == 0:
                @pl.when(up > 0)
                def _():
                    drain_scatters(1 - q, 1, 3 * NBUF)
                    fire_super(up * 2 + 1, 1)
            elif j == 4:
                qp, pp, rp = grp(3)
                drain_scatters(qp, pp, rp)

                @pl.when(up < NSUP // 2 - 1)
                def _():
                    fire_super(up * 2 + 2, 0)
            else:
                qp, pp, rp = grp(j - 1)
                drain_scatters(qp, pp, rp)
            qn, pn, rn = grp((j + 1) % 8)
            if j == 7:
                @pl.when(up < NSUP // 2 - 1)
                def _():
                    drain_super(0)
                    fire_gathers(qn, pn, rn)
            else:
                if j == 3:
                    drain_super(1)
                fire_gathers(qn, pn, rn)
            fire_scatters(q, p, r0)
        return 0

    lax.fori_loop(0, NSUP // 2, body, 0)

    q, p, r0 = grp(7)
    drain_scatters(q, p, r0)

    plsc.subcore_barrier()
    pltpu.sync_copy(stab.at[pl.ds(s * STRIPE, STRIPE)],
                    sout.at[c, pl.ds(s * STRIPE, STRIPE), :])


def _scale_layer_body(s_ref, dinv_ref, dinv2_ref, accp_ref, y_ref, acc_ref):
    for h in range(2):
        sv = s_ref[h]
        y_ref[h] = sv * dinv2_ref[...]
        acc_ref[h] = accp_ref[h] + sv * dinv_ref[...]


def _scale_layer(sout, dinv, dinv2, acc_prev):
    nb = 16
    rb = NPAD // nb
    return pl.pallas_call(
        _scale_layer_body,
        grid=(nb,),
        in_specs=[
            pl.BlockSpec((2, rb, DH), lambda r: (0, r, 0)),
            pl.BlockSpec((rb, 1), lambda r: (r, 0)),
            pl.BlockSpec((rb, 1), lambda r: (r, 0)),
            pl.BlockSpec((2, rb, DH), lambda r: (0, r, 0)),
        ],
        out_specs=[
            pl.BlockSpec((2, rb, DH), lambda r: (0, r, 0)),
            pl.BlockSpec((2, rb, DH), lambda r: (0, r, 0)),
        ],
        out_shape=[
            jax.ShapeDtypeStruct((2, NPAD, DH), jnp.float32),
            jax.ShapeDtypeStruct((2, NPAD, DH), jnp.float32),
        ],
    )(sout, dinv, dinv2, acc_prev)


PER_TILE_B = BATCH // 32


@functools.partial(
    pl.kernel,
    out_type=(
        jax.ShapeDtypeStruct((2, 3, BATCH, DH), jnp.float32),
        jax.ShapeDtypeStruct((3, BATCH, D), jnp.float32),
    ),
    mesh=_mesh,
    compiler_params=pltpu.CompilerParams(use_tc_tiling_on_sc=False),
    scratch_types=(
        pltpu.VMEM((2, PER_TILE_B), jnp.int32),
        pltpu.VMEM((2, PER_TILE_B, DH), jnp.float32),
        pltpu.VMEM((PER_TILE_B, D), jnp.float32),
        pltpu.SemaphoreType.DMA,
    ),
)
def _batch_gather_kernel(acc, emb, idx2, accg, egog, idxb, rbuf, ebuf, sem):
    c = lax.axis_index("c")
    s = lax.axis_index("s")
    wid = s * 2 + c
    for a in range(3):
        for h in range(2):
            pltpu.sync_copy(
                idx2.at[h, a, pl.ds(wid * PER_TILE_B, PER_TILE_B)],
                idxb.at[h])
        for j in range(2):
            for h in range(2):
                pltpu.async_copy(acc.at[idxb.at[h, pl.ds(j * 128, 128)]],
                                 rbuf.at[h, pl.ds(j * 128, 128), :], sem)
            pltpu.async_copy(emb.at[idxb.at[0, pl.ds(j * 128, 128)]],
                             ebuf.at[pl.ds(j * 128, 128), :], sem)
        for j in range(2):
            for h in range(2):
                pltpu.make_async_copy(
                    acc.at[idxb.at[h, pl.ds(j * 128, 128)]],
                    rbuf.at[h, pl.ds(j * 128, 128), :], sem).wait()
            pltpu.make_async_copy(emb.at[idxb.at[0, pl.ds(j * 128, 128)]],
                                  ebuf.at[pl.ds(j * 128, 128), :], sem).wait()
        for h in range(2):
            pltpu.sync_copy(
                rbuf.at[h],
                accg.at[h, a, pl.ds(wid * PER_TILE_B, PER_TILE_B), :])
        pltpu.sync_copy(ebuf, egog.at[a, pl.ds(wid * PER_TILE_B, PER_TILE_B), :])


def _loss_body(accg_ref, egog_ref, out_ref):
    u = accg_ref[:, 0]
    p = accg_ref[:, 1]
    n = accg_ref[:, 2]
    pos = jnp.sum(jnp.sum(u * p, axis=-1), axis=0)
    neg = jnp.sum(jnp.sum(u * n, axis=-1), axis=0)
    x = (neg - pos) * 0.0625
    cf = jnp.mean(jnp.maximum(x, 0.0) + jnp.log1p(jnp.exp(-jnp.abs(x))))
    e = egog_ref[...]
    reg = 0.5 * jnp.sum(e * e) / float(BATCH)
    out_ref[...] = jnp.reshape(CF_WEIGHT * cf + reg * WEIGHT_DECAY, (1, 1))


def _loss(accg, egog):
    return pl.pallas_call(
        _loss_body,
        out_shape=jax.ShapeDtypeStruct((1, 1), jnp.float32),
    )(accg, egog)


def kernel(user_idx, pos_item, neg_item, edge_index, emb_weight):
    edge_index = edge_index.astype(jnp.int32)
    npad_e = NE_PAD - N_EDGES
    fr = (jnp.arange(npad_e, dtype=jnp.int32) * 37) % N_NODES
    fc = N_NODES + (jnp.arange(npad_e, dtype=jnp.int32) % (NPAD - N_NODES))
    row3 = jnp.concatenate([edge_index[0], fr]).reshape(N_CHUNK_ROWS, CHUNK)
    col3 = jnp.concatenate([edge_index[1], fc]).reshape(N_CHUNK_ROWS, CHUNK)
    ridx2 = jnp.stack([row3, row3 + NPAD])
    emb_pad = jnp.pad(emb_weight, ((0, NPAD - N_NODES), (0, 0)))
    idxs = jnp.stack([user_idx, pos_item, neg_item]).astype(jnp.int32)
    idx2 = jnp.stack([idxs, idxs + NPAD])

    degp = _deg_kernel(col3)
    dinv, dinv2, y2, acc2 = _scale_init(degp, emb_pad)
    for _ in range(N_LAYERS):
        sout = _layer_kernel(y2.reshape(2 * NPAD, DH), ridx2, col3)
        y2, acc2 = _scale_layer(sout, dinv, dinv2, acc2)

    accg, egog = _batch_gather_kernel(acc2.reshape(2 * NPAD, DH),
                                      emb_weight, idx2)
    loss = _loss(accg, egog)
    return jnp.reshape(loss, ())


# speedup vs baseline: 18.0115x; 1.2651x over previous
import functools

import jax
import jax.numpy as jnp
from jax import lax
from jax.experimental import pallas as pl
from jax.experimental.pallas import tpu as pltpu
from jax.experimental.pallas import tpu_sc as plsc

N_NODES = 50000
N_USERS = 10000
N_EDGES = 800000
D = 64
BATCH = 8192
N_LAYERS = 3
CF_WEIGHT = 1.0
WEIGHT_DECAY = 1e-4

NPAD = 50048
DH = 32
CHUNK = 80
NE_PAD = 819200
N_CHUNK_ROWS = NE_PAD // CHUNK

_mesh = plsc.VectorSubcoreMesh(core_axis_name="c", subcore_axis_name="s")


def _fill_zeros(ref, nrows, ncols):
    zero16 = jnp.zeros((16,), jnp.float32)

    def body(i, _):
        for k in range(ncols // 16):
            ref[i, pl.ds(k * 16, 16)] = zero16
        return 0

    lax.fori_loop(0, nrows, body, 0)


ROWS_PER_TILE_A = N_CHUNK_ROWS // 32
GROUPS_A = 10
ROWS_PER_GROUP_A = ROWS_PER_TILE_A // GROUPS_A


@functools.partial(
    pl.kernel,
    out_type=jax.ShapeDtypeStruct((2, NPAD, 16), jnp.float32),
    mesh=_mesh,
    compiler_params=pltpu.CompilerParams(use_tc_tiling_on_sc=False),
    scratch_types=(
        pltpu.VMEM((ROWS_PER_GROUP_A, CHUNK), jnp.int32),
        pltpu.VMEM((CHUNK, 16), jnp.float32),
        pltpu.VMEM((3128, 16), jnp.float32),
        pltpu.VMEM_SHARED((NPAD, 16), jnp.float32),
        pltpu.SemaphoreType.DMA,
    ),
)
def _deg_kernel(col3, degp, cbuf, ones, zb, degtab, dsem):
    c = lax.axis_index("c")
    s = lax.axis_index("s")
    wid = s * 2 + c

    _fill_zeros(zb, 3128, 16)
    pltpu.sync_copy(zb, degtab.at[pl.ds(s * 3128, 3128)])
    plsc.subcore_barrier()

    one16 = jnp.full((16,), 1.0, jnp.float32)

    def fill_ones(i, _):
        ones[i] = one16
        return 0

    lax.fori_loop(0, CHUNK, fill_ones, 0)

    def do_group(r0, nrows):
        pltpu.sync_copy(col3.at[pl.ds(r0, nrows)],
                        cbuf.at[pl.ds(0, nrows)])
        for r in range(nrows):
            pltpu.async_copy(ones, degtab.at[cbuf.at[r]], dsem, add=True)
        for r in range(nrows):
            pltpu.make_async_copy(ones, degtab.at[cbuf.at[r]], dsem).wait()

    def group_body(g, _):
        do_group(wid * ROWS_PER_TILE_A + g * ROWS_PER_GROUP_A,
                 ROWS_PER_GROUP_A)
        return 0

    lax.fori_loop(0, GROUPS_A, group_body, 0)

    plsc.subcore_barrier()
    pltpu.sync_copy(degtab.at[pl.ds(s * 3128, 3128)],
                    degp.at[c, pl.ds(s * 3128, 3128), :])


def _scale_init_body(degp_ref, emb_ref, d1rep_ref, d2rep_ref, y0_ref,
                     acc0_ref):
    deg = degp_ref[0, :, 0:1] + degp_ref[1, :, 0:1]
    dinv = jnp.where(deg > 0, lax.rsqrt(jnp.maximum(deg, 1e-12)), 0.0)
    ev = emb_ref[...]
    for h in range(2):
        d1rep_ref[h] = jnp.broadcast_to(dinv, dinv.shape[:1] + (DH,))
        d2rep_ref[h] = jnp.broadcast_to(dinv * dinv,
                                        dinv.shape[:1] + (DH,))
        y0_ref[h] = ev[:, h * DH:(h + 1) * DH] * dinv
        acc0_ref[h] = ev[:, h * DH:(h + 1) * DH]


def _scale_init(degp, emb_pad):
    nb = 16
    rb = NPAD // nb
    return pl.pallas_call(
        _scale_init_body,
        grid=(nb,),
        in_specs=[
            pl.BlockSpec((2, rb, 16), lambda r: (0, r, 0)),
            pl.BlockSpec((rb, D), lambda r: (r, 0)),
        ],
        out_specs=[
            pl.BlockSpec((2, rb, DH), lambda r: (0, r, 0)),
            pl.BlockSpec((2, rb, DH), lambda r: (0, r, 0)),
            pl.BlockSpec((2, rb, DH), lambda r: (0, r, 0)),
            pl.BlockSpec((2, rb, DH), lambda r: (0, r, 0)),
        ],
        out_shape=[
            jax.ShapeDtypeStruct((2, NPAD, DH), jnp.float32),
            jax.ShapeDtypeStruct((2, NPAD, DH), jnp.float32),
            jax.ShapeDtypeStruct((2, NPAD, DH), jnp.float32),
            jax.ShapeDtypeStruct((2, NPAD, DH), jnp.float32),
        ],
    )(degp, emb_pad)


ROWS_PER_TILE_C = N_CHUNK_ROWS // 16
NBUF = 2
SUP = 8
GPS = SUP // NBUF
NSUP = ROWS_PER_TILE_C // SUP
NG = ROWS_PER_TILE_C // NBUF
STRIPE = NPAD // 16


@functools.partial(
    pl.kernel,
    out_type=jax.ShapeDtypeStruct((2, NPAD, DH), jnp.float32),
    mesh=_mesh,
    compiler_params=pltpu.CompilerParams(use_tc_tiling_on_sc=False),
    scratch_types=(
        pltpu.VMEM((2, NBUF * CHUNK, DH), jnp.float32),
        pltpu.VMEM((2, SUP, CHUNK), jnp.int32),
        pltpu.VMEM((2, SUP, CHUNK), jnp.int32),
        pltpu.VMEM_SHARED((NPAD, DH), jnp.float32),
        pltpu.SemaphoreType.DMA,
        pltpu.SemaphoreType.DMA,
        pltpu.SemaphoreType.DMA,
        pltpu.SemaphoreType.DMA,
    ),
)
def _layer_kernel(ycat, ridx2, col3, sout, rows, idxr, idxc, stab,
                  gsem, ssem, isem0, isem1):
    c = lax.axis_index("c")
    s = lax.axis_index("s")
    base_row = s * ROWS_PER_TILE_C

    zrows = NBUF * CHUNK
    _fill_zeros(rows.at[0], zrows, DH)

    def zbody(k, _):
        pltpu.sync_copy(rows.at[0], stab.at[pl.ds(s * STRIPE + k * zrows,
                                                  zrows)])
        return 0

    lax.fori_loop(0, STRIPE // zrows, zbody, 0)
    rem = STRIPE % zrows
    pltpu.sync_copy(rows.at[0, pl.ds(0, rem), :],
                    stab.at[pl.ds(s * STRIPE + STRIPE - rem, rem)])
    plsc.subcore_barrier()

    isems = (isem0, isem1)

    def fire_super(u, parity):
        pltpu.async_copy(ridx2.at[c, pl.ds(base_row + u * SUP, SUP), :],
                         idxr.at[parity], isems[parity])
        pltpu.async_copy(col3.at[pl.ds(base_row + u * SUP, SUP)],
                         idxc.at[parity], isems[parity])

    def drain_super(parity):
        pltpu.make_async_copy(ridx2.at[0, pl.ds(0, SUP), :],
                              idxr.at[parity], isems[parity]).wait()
        pltpu.make_async_copy(col3.at[pl.ds(0, SUP)],
                              idxc.at[parity], isems[parity]).wait()

    def fire_gathers(q, p, r0):
        for b in range(NBUF):
            pltpu.async_copy(ycat.at[idxr.at[p, r0 + b]],
                             rows.at[q, pl.ds(b * CHUNK, CHUNK), :],
                             gsem)

    def drain_gathers(q, p, r0):
        for b in range(NBUF):
            pltpu.make_async_copy(
                ycat.at[idxr.at[p, r0 + b]],
                rows.at[q, pl.ds(b * CHUNK, CHUNK), :], gsem).wait()

    def fire_scatters(q, p, r0):
        for b in range(NBUF):
            pltpu.async_copy(rows.at[q, pl.ds(b * CHUNK, CHUNK), :],
                             stab.at[idxc.at[p, r0 + b]], ssem, add=True)

    def drain_scatters(q, p, r0):
        for b in range(NBUF):
            pltpu.make_async_copy(
                rows.at[q, pl.ds(b * CHUNK, CHUNK), :],
                stab.at[idxc.at[p, r0 + b]], ssem).wait()

    def grp(j):
        return (j % 2, (j // 4) % 2, (j % 4) * NBUF)

    fire_super(0, 0)
    fire_super(1, 1)
    drain_super(0)
    fire_gathers(0, 0, 0)

    def body(up, _):
        for j in range(8):
            q, p, r0 = grp(j)
            drain_gathers(q, p, r0)
            if j == 0:
                @pl.when(up > 0)
                def _():
                    drain_scatters(1 - q, 1, 3 * NBUF)
                    fire_super(up * 2 + 1, 1)
            elif j == 4:
                qp, pp, rp = grp(3)
                drain_scatters(qp, pp, rp)

                @pl.when(up < NSUP // 2 - 1)
                def _():
                    fire_super(up * 2 + 2, 0)
            else:
                qp, pp, rp = grp(j - 1)
                drain_scatters(qp, pp, rp)
            qn, pn, rn = grp((j + 1) % 8)
            if j == 7:
                @pl.when(up < NSUP // 2 - 1)
                def _():
                    drain_super(0)
                    fire_gathers(qn, pn, rn)
            else:
                if j == 3:
                    drain_super(1)
                fire_gathers(qn, pn, rn)
            fire_scatters(q, p, r0)
        return 0

    lax.fori_loop(0, NSUP // 2, body, 0)

    q, p, r0 = grp(7)
    drain_scatters(q, p, r0)

    plsc.subcore_barrier()
    pltpu.sync_copy(stab.at[pl.ds(s * STRIPE, STRIPE)],
                    sout.at[c, pl.ds(s * STRIPE, STRIPE), :])


FLAT_R = 2 * NPAD * DH // 128


def _scale_layer_body(s_ref, d1_ref, d2_ref, accp_ref, y_ref, acc_ref):
    sv = s_ref[...]
    y_ref[...] = sv * d2_ref[...]
    acc_ref[...] = accp_ref[...] + sv * d1_ref[...]


def _scale_layer(sout_f, d1rep_f, d2rep_f, accp_f):
    nb = 8
    rb = FLAT_R // nb
    spec = pl.BlockSpec((rb, 128), lambda r: (r, 0))
    return pl.pallas_call(
        _scale_layer_body,
        grid=(nb,),
        in_specs=[spec, spec, spec, spec],
        out_specs=[spec, spec],
        out_shape=[
            jax.ShapeDtypeStruct((FLAT_R, 128), jnp.float32),
            jax.ShapeDtypeStruct((FLAT_R, 128), jnp.float32),
        ],
    )(sout_f, d1rep_f, d2rep_f, accp_f)


PER_TILE_B = BATCH // 32


@functools.partial(
    pl.kernel,
    out_type=(
        jax.ShapeDtypeStruct((2, 3, BATCH, DH), jnp.float32),
        jax.ShapeDtypeStruct((3, BATCH, D), jnp.float32),
    ),
    mesh=_mesh,
    compiler_params=pltpu.CompilerParams(use_tc_tiling_on_sc=False),
    scratch_types=(
        pltpu.VMEM((2, PER_TILE_B), jnp.int32),
        pltpu.VMEM((2, PER_TILE_B, DH), jnp.float32),
        pltpu.VMEM((PER_TILE_B, D), jnp.float32),
        pltpu.SemaphoreType.DMA,
    ),
)
def _batch_gather_kernel(acc, emb, idx2, accg, egog, idxb, rbuf, ebuf, sem):
    c = lax.axis_index("c")
    s = lax.axis_index("s")
    wid = s * 2 + c
    for a in range(3):
        for h in range(2):
            pltpu.sync_copy(
                idx2.at[h, a, pl.ds(wid * PER_TILE_B, PER_TILE_B)],
                idxb.at[h])
        for j in range(2):
            for h in range(2):
                pltpu.async_copy(acc.at[idxb.at[h, pl.ds(j * 128, 128)]],
                                 rbuf.at[h, pl.ds(j * 128, 128), :], sem)
            pltpu.async_copy(emb.at[idxb.at[0, pl.ds(j * 128, 128)]],
                             ebuf.at[pl.ds(j * 128, 128), :], sem)
        for j in range(2):
            for h in range(2):
                pltpu.make_async_copy(
                    acc.at[idxb.at[h, pl.ds(j * 128, 128)]],
                    rbuf.at[h, pl.ds(j * 128, 128), :], sem).wait()
            pltpu.make_async_copy(emb.at[idxb.at[0, pl.ds(j * 128, 128)]],
                                  ebuf.at[pl.ds(j * 128, 128), :], sem).wait()
        for h in range(2):
            pltpu.sync_copy(
                rbuf.at[h],
                accg.at[h, a, pl.ds(wid * PER_TILE_B, PER_TILE_B), :])
        pltpu.sync_copy(ebuf, egog.at[a, pl.ds(wid * PER_TILE_B, PER_TILE_B), :])


def _loss_body(accg_ref, egog_ref, out_ref):
    u = accg_ref[:, 0]
    p = accg_ref[:, 1]
    n = accg_ref[:, 2]
    pos = jnp.sum(jnp.sum(u * p, axis=-1), axis=0)
    neg = jnp.sum(jnp.sum(u * n, axis=-1), axis=0)
    x = (neg - pos) * 0.0625
    cf = jnp.mean(jnp.maximum(x, 0.0) + jnp.log1p(jnp.exp(-jnp.abs(x))))
    e = egog_ref[...]
    reg = 0.5 * jnp.sum(e * e) / float(BATCH)
    out_ref[...] = jnp.reshape(CF_WEIGHT * cf + reg * WEIGHT_DECAY, (1, 1))


def _loss(accg, egog):
    return pl.pallas_call(
        _loss_body,
        out_shape=jax.ShapeDtypeStruct((1, 1), jnp.float32),
    )(accg, egog)


def kernel(user_idx, pos_item, neg_item, edge_index, emb_weight):
    edge_index = edge_index.astype(jnp.int32)
    npad_e = NE_PAD - N_EDGES
    fr = (jnp.arange(npad_e, dtype=jnp.int32) * 37) % N_NODES
    fc = N_NODES + (jnp.arange(npad_e, dtype=jnp.int32) % (NPAD - N_NODES))
    row3 = jnp.concatenate([edge_index[0], fr]).reshape(N_CHUNK_ROWS, CHUNK)
    col3 = jnp.concatenate([edge_index[1], fc]).reshape(N_CHUNK_ROWS, CHUNK)
    ridx2 = jnp.stack([row3, row3 + NPAD])
    emb_pad = jnp.pad(emb_weight, ((0, NPAD - N_NODES), (0, 0)))
    idxs = jnp.stack([user_idx, pos_item, neg_item]).astype(jnp.int32)
    idx2 = jnp.stack([idxs, idxs + NPAD])

    degp = _deg_kernel(col3)
    d1rep, d2rep, y2, acc2 = _scale_init(degp, emb_pad)
    d1f = d1rep.reshape(FLAT_R, 128)
    d2f = d2rep.reshape(FLAT_R, 128)
    ycat = y2.reshape(2 * NPAD, DH)
    accf = acc2.reshape(FLAT_R, 128)
    for _ in range(N_LAYERS):
        sout = _layer_kernel(ycat, ridx2, col3)
        yf, accf = _scale_layer(sout.reshape(FLAT_R, 128), d1f, d2f, accf)
        ycat = yf.reshape(2 * NPAD, DH)

    accg, egog = _batch_gather_kernel(accf.reshape(2 * NPAD, DH),
                                      emb_weight, idx2)
    loss = _loss(accg, egog)
    return jnp.reshape(loss, ())


# speedup vs baseline: 23.1209x; 1.2837x over previous
import functools

import jax
import jax.numpy as jnp
from jax import lax
from jax.experimental import pallas as pl
from jax.experimental.pallas import tpu as pltpu
from jax.experimental.pallas import tpu_sc as plsc

N_NODES = 50000
N_USERS = 10000
N_EDGES = 800000
D = 64
BATCH = 8192
N_LAYERS = 3
CF_WEIGHT = 1.0
WEIGHT_DECAY = 1e-4

NPAD = 50048
DH = 32
CHUNK = 80
NE_PAD = 819200
N_CHUNK_ROWS = NE_PAD // CHUNK

_mesh = plsc.VectorSubcoreMesh(core_axis_name="c", subcore_axis_name="s")


def _fill_zeros(ref, nrows, ncols):
    zero16 = jnp.zeros((16,), jnp.float32)

    def body(i, _):
        for k in range(ncols // 16):
            ref[i, pl.ds(k * 16, 16)] = zero16
        return 0

    lax.fori_loop(0, nrows, body, 0)


ROWS_PER_TILE_A = N_CHUNK_ROWS // 32
GROUPS_A = 10
ROWS_PER_GROUP_A = ROWS_PER_TILE_A // GROUPS_A


@functools.partial(
    pl.kernel,
    out_type=jax.ShapeDtypeStruct((2, NPAD, 16), jnp.float32),
    mesh=_mesh,
    compiler_params=pltpu.CompilerParams(use_tc_tiling_on_sc=False),
    scratch_types=(
        pltpu.VMEM((ROWS_PER_GROUP_A, CHUNK), jnp.int32),
        pltpu.VMEM((CHUNK, 16), jnp.float32),
        pltpu.VMEM((3128, 16), jnp.float32),
        pltpu.VMEM_SHARED((NPAD, 16), jnp.float32),
        pltpu.SemaphoreType.DMA,
    ),
)
def _deg_kernel(col3, degp, cbuf, ones, zb, degtab, dsem):
    c = lax.axis_index("c")
    s = lax.axis_index("s")
    wid = s * 2 + c

    _fill_zeros(zb, 3128, 16)
    pltpu.sync_copy(zb, degtab.at[pl.ds(s * 3128, 3128)])
    plsc.subcore_barrier()

    one16 = jnp.full((16,), 1.0, jnp.float32)

    def fill_ones(i, _):
        ones[i] = one16
        return 0

    lax.fori_loop(0, CHUNK, fill_ones, 0)

    def do_group(r0, nrows):
        pltpu.sync_copy(col3.at[pl.ds(r0, nrows)],
                        cbuf.at[pl.ds(0, nrows)])
        for r in range(nrows):
            pltpu.async_copy(ones, degtab.at[cbuf.at[r]], dsem, add=True)
        for r in range(nrows):
            pltpu.make_async_copy(ones, degtab.at[cbuf.at[r]], dsem).wait()

    def group_body(g, _):
        do_group(wid * ROWS_PER_TILE_A + g * ROWS_PER_GROUP_A,
                 ROWS_PER_GROUP_A)
        return 0

    lax.fori_loop(0, GROUPS_A, group_body, 0)

    plsc.subcore_barrier()
    pltpu.sync_copy(degtab.at[pl.ds(s * 3128, 3128)],
                    degp.at[c, pl.ds(s * 3128, 3128), :])


def _scale_init_body(degp_ref, emb_ref, d1rep_ref, d2rep_ref, y0_ref,
                     acc0_ref):
    deg = degp_ref[0, :, 0:1] + degp_ref[1, :, 0:1]
    dinv = jnp.where(deg > 0, lax.rsqrt(jnp.maximum(deg, 1e-12)), 0.0)
    ev = emb_ref[...]
    for h in range(2):
        d1rep_ref[h] = jnp.broadcast_to(dinv, dinv.shape[:1] + (DH,))
        d2rep_ref[h] = jnp.broadcast_to(dinv * dinv,
                                        dinv.shape[:1] + (DH,))
        y0_ref[h] = ev[:, h * DH:(h + 1) * DH] * dinv
        acc0_ref[h] = ev[:, h * DH:(h + 1) * DH]


def _scale_init(degp, emb_pad):
    nb = 16
    rb = NPAD // nb
    return pl.pallas_call(
        _scale_init_body,
        grid=(nb,),
        in_specs=[
            pl.BlockSpec((2, rb, 16), lambda r: (0, r, 0)),
            pl.BlockSpec((rb, D), lambda r: (r, 0)),
        ],
        out_specs=[
            pl.BlockSpec((2, rb, DH), lambda r: (0, r, 0)),
            pl.BlockSpec((2, rb, DH), lambda r: (0, r, 0)),
            pl.BlockSpec((2, rb, DH), lambda r: (0, r, 0)),
            pl.BlockSpec((2, rb, DH), lambda r: (0, r, 0)),
        ],
        out_shape=[
            jax.ShapeDtypeStruct((2, NPAD, DH), jnp.float32),
            jax.ShapeDtypeStruct((2, NPAD, DH), jnp.float32),
            jax.ShapeDtypeStruct((2, NPAD, DH), jnp.float32),
            jax.ShapeDtypeStruct((2, NPAD, DH), jnp.float32),
        ],
    )(degp, emb_pad)


ROWS_PER_TILE_C = N_CHUNK_ROWS // 16
NBUF = 4
SUP = 8
GPS = SUP // NBUF
NSUP = ROWS_PER_TILE_C // SUP
NG = ROWS_PER_TILE_C // NBUF
GPI = 2 * GPS
STRIPE = NPAD // 16


@functools.partial(
    pl.kernel,
    out_type=jax.ShapeDtypeStruct((2, NPAD, DH), jnp.float32),
    mesh=_mesh,
    compiler_params=pltpu.CompilerParams(use_tc_tiling_on_sc=False),
    scratch_types=(
        pltpu.VMEM((2, NBUF * CHUNK, DH), jnp.float32),
        pltpu.VMEM((2, SUP, CHUNK), jnp.int32),
        pltpu.VMEM((2, SUP, CHUNK), jnp.int32),
        pltpu.VMEM_SHARED((NPAD, DH), jnp.float32),
        pltpu.SemaphoreType.DMA,
        pltpu.SemaphoreType.DMA,
        pltpu.SemaphoreType.DMA,
        pltpu.SemaphoreType.DMA,
    ),
)
def _layer_kernel(ycat, ridx2, col3, sout, rows, idxr, idxc, stab,
                  gsem, ssem, isem0, isem1):
    c = lax.axis_index("c")
    s = lax.axis_index("s")
    base_row = s * ROWS_PER_TILE_C

    zrows = NBUF * CHUNK
    _fill_zeros(rows.at[0], zrows, DH)

    def zbody(k, _):
        pltpu.sync_copy(rows.at[0], stab.at[pl.ds(s * STRIPE + k * zrows,
                                                  zrows)])
        return 0

    lax.fori_loop(0, STRIPE // zrows, zbody, 0)
    rem = STRIPE % zrows
    pltpu.sync_copy(rows.at[0, pl.ds(0, rem), :],
                    stab.at[pl.ds(s * STRIPE + STRIPE - rem, rem)])
    plsc.subcore_barrier()

    isems = (isem0, isem1)

    def fire_super(u, parity):
        pltpu.async_copy(ridx2.at[c, pl.ds(base_row + u * SUP, SUP), :],
                         idxr.at[parity], isems[parity])
        pltpu.async_copy(col3.at[pl.ds(base_row + u * SUP, SUP)],
                         idxc.at[parity], isems[parity])

    def drain_super(parity):
        pltpu.make_async_copy(ridx2.at[0, pl.ds(0, SUP), :],
                              idxr.at[parity], isems[parity]).wait()
        pltpu.make_async_copy(col3.at[pl.ds(0, SUP)],
                              idxc.at[parity], isems[parity]).wait()

    def fire_gathers(q, p, r0):
        for b in range(NBUF):
            pltpu.async_copy(ycat.at[idxr.at[p, r0 + b]],
                             rows.at[q, pl.ds(b * CHUNK, CHUNK), :],
                             gsem)

    def drain_gathers(q, p, r0):
        for b in range(NBUF):
            pltpu.make_async_copy(
                ycat.at[idxr.at[p, r0 + b]],
                rows.at[q, pl.ds(b * CHUNK, CHUNK), :], gsem).wait()

    def fire_scatters(q, p, r0):
        for b in range(NBUF):
            pltpu.async_copy(rows.at[q, pl.ds(b * CHUNK, CHUNK), :],
                             stab.at[idxc.at[p, r0 + b]], ssem, add=True)

    def drain_scatters(q, p, r0):
        for b in range(NBUF):
            pltpu.make_async_copy(
                rows.at[q, pl.ds(b * CHUNK, CHUNK), :],
                stab.at[idxc.at[p, r0 + b]], ssem).wait()

    def grp(j):
        return (j % 2, (j // GPS) % 2, (j % GPS) * NBUF)

    fire_super(0, 0)
    fire_super(1, 1)
    drain_super(0)
    fire_gathers(0, 0, 0)

    def body(up, _):
        for j in range(GPI):
            q, p, r0 = grp(j)
            drain_gathers(q, p, r0)
            if j == 0:
                @pl.when(up > 0)
                def _():
                    qp, pp, rp = grp(GPI - 1)
                    drain_scatters(qp, pp, rp)
                    fire_super(up * 2 + 1, 1)
            elif j == GPS:
                qp, pp, rp = grp(GPS - 1)
                drain_scatters(qp, pp, rp)

                @pl.when(up < NSUP // 2 - 1)
                def _():
                    fire_super(up * 2 + 2, 0)
            else:
                qp, pp, rp = grp(j - 1)
                drain_scatters(qp, pp, rp)
            qn, pn, rn = grp((j + 1) % GPI)
            if j == GPI - 1:
                @pl.when(up < NSUP // 2 - 1)
                def _():
                    drain_super(0)
                    fire_gathers(qn, pn, rn)
            else:
                if j == GPS - 1:
                    drain_super(1)
                fire_gathers(qn, pn, rn)
            fire_scatters(q, p, r0)
        return 0

    lax.fori_loop(0, NSUP // 2, body, 0)

    q, p, r0 = grp(GPI - 1)
    drain_scatters(q, p, r0)

    plsc.subcore_barrier()
    pltpu.sync_copy(stab.at[pl.ds(s * STRIPE, STRIPE)],
                    sout.at[c, pl.ds(s * STRIPE, STRIPE), :])


FLAT_R = 2 * NPAD * DH // 128


def _scale_layer_body(s_ref, d1_ref, d2_ref, accp_ref, y_ref, acc_ref):
    sv = s_ref[...]
    y_ref[...] = sv * d2_ref[...]
    acc_ref[...] = accp_ref[...] + sv * d1_ref[...]


def _scale_layer(sout_f, d1rep_f, d2rep_f, accp_f):
    nb = 8
    rb = FLAT_R // nb
    spec = pl.BlockSpec((rb, 128), lambda r: (r, 0))
    return pl.pallas_call(
        _scale_layer_body,
        grid=(nb,),
        in_specs=[spec, spec, spec, spec],
        out_specs=[spec, spec],
        out_shape=[
            jax.ShapeDtypeStruct((FLAT_R, 128), jnp.float32),
            jax.ShapeDtypeStruct((FLAT_R, 128), jnp.float32),
        ],
    )(sout_f, d1rep_f, d2rep_f, accp_f)


PER_TILE_B = BATCH // 32


@functools.partial(
    pl.kernel,
    out_type=(
        jax.ShapeDtypeStruct((2, 3, BATCH, DH), jnp.float32),
        jax.ShapeDtypeStruct((3, BATCH, D), jnp.float32),
    ),
    mesh=_mesh,
    compiler_params=pltpu.CompilerParams(use_tc_tiling_on_sc=False),
    scratch_types=(
        pltpu.VMEM((2, PER_TILE_B), jnp.int32),
        pltpu.VMEM((2, PER_TILE_B, DH), jnp.float32),
        pltpu.VMEM((PER_TILE_B, D), jnp.float32),
        pltpu.SemaphoreType.DMA,
    ),
)
def _batch_gather_kernel(acc, emb, idx2, accg, egog, idxb, rbuf, ebuf, sem):
    c = lax.axis_index("c")
    s = lax.axis_index("s")
    wid = s * 2 + c
    for a in range(3):
        for h in range(2):
            pltpu.sync_copy(
                idx2.at[h, a, pl.ds(wid * PER_TILE_B, PER_TILE_B)],
                idxb.at[h])
        for j in range(2):
            for h in range(2):
                pltpu.async_copy(acc.at[idxb.at[h, pl.ds(j * 128, 128)]],
                                 rbuf.at[h, pl.ds(j * 128, 128), :], sem)
            pltpu.async_copy(emb.at[idxb.at[0, pl.ds(j * 128, 128)]],
                             ebuf.at[pl.ds(j * 128, 128), :], sem)
        for j in range(2):
            for h in range(2):
                pltpu.make_async_copy(
                    acc.at[idxb.at[h, pl.ds(j * 128, 128)]],
                    rbuf.at[h, pl.ds(j * 128, 128), :], sem).wait()
            pltpu.make_async_copy(emb.at[idxb.at[0, pl.ds(j * 128, 128)]],
                                  ebuf.at[pl.ds(j * 128, 128), :], sem).wait()
        for h in range(2):
            pltpu.sync_copy(
                rbuf.at[h],
                accg.at[h, a, pl.ds(wid * PER_TILE_B, PER_TILE_B), :])
        pltpu.sync_copy(ebuf, egog.at[a, pl.ds(wid * PER_TILE_B, PER_TILE_B), :])


def _loss_body(accg_ref, egog_ref, out_ref):
    u = accg_ref[:, 0]
    p = accg_ref[:, 1]
    n = accg_ref[:, 2]
    pos = jnp.sum(jnp.sum(u * p, axis=-1), axis=0)
    neg = jnp.sum(jnp.sum(u * n, axis=-1), axis=0)
    x = (neg - pos) * 0.0625
    cf = jnp.mean(jnp.maximum(x, 0.0) + jnp.log1p(jnp.exp(-jnp.abs(x))))
    e = egog_ref[...]
    reg = 0.5 * jnp.sum(e * e) / float(BATCH)
    out_ref[...] = jnp.reshape(CF_WEIGHT * cf + reg * WEIGHT_DECAY, (1, 1))


def _loss(accg, egog):
    return pl.pallas_call(
        _loss_body,
        out_shape=jax.ShapeDtypeStruct((1, 1), jnp.float32),
    )(accg, egog)


def kernel(user_idx, pos_item, neg_item, edge_index, emb_weight):
    edge_index = edge_index.astype(jnp.int32)
    npad_e = NE_PAD - N_EDGES
    fr = (jnp.arange(npad_e, dtype=jnp.int32) * 37) % N_NODES
    fc = N_NODES + (jnp.arange(npad_e, dtype=jnp.int32) % (NPAD - N_NODES))
    row3 = jnp.concatenate([edge_index[0], fr]).reshape(N_CHUNK_ROWS, CHUNK)
    col3 = jnp.concatenate([edge_index[1], fc]).reshape(N_CHUNK_ROWS, CHUNK)
    ridx2 = jnp.stack([row3, row3 + NPAD])
    emb_pad = jnp.pad(emb_weight, ((0, NPAD - N_NODES), (0, 0)))
    idxs = jnp.stack([user_idx, pos_item, neg_item]).astype(jnp.int32)
    idx2 = jnp.stack([idxs, idxs + NPAD])

    degp = _deg_kernel(col3)
    d1rep, d2rep, y2, acc2 = _scale_init(degp, emb_pad)
    d1f = d1rep.reshape(FLAT_R, 128)
    d2f = d2rep.reshape(FLAT_R, 128)
    ycat = y2.reshape(2 * NPAD, DH)
    accf = acc2.reshape(FLAT_R, 128)
    for _ in range(N_LAYERS):
        sout = _layer_kernel(ycat, ridx2, col3)
        yf, accf = _scale_layer(sout.reshape(FLAT_R, 128), d1f, d2f, accf)
        ycat = yf.reshape(2 * NPAD, DH)

    accg, egog = _batch_gather_kernel(accf.reshape(2 * NPAD, DH),
                                      emb_weight, idx2)
    loss = _loss(accg, egog)
    return jnp.reshape(loss, ())
